# baseline (speedup 1.0000x reference)
import functools

import jax
import jax.numpy as jnp
from jax import lax
from jax.experimental import pallas as pl
from jax.experimental.pallas import tpu as pltpu
from jax.experimental.pallas import tpu_sc as plsc

B = 16384
D = 64

_NC = 2
_NS = 16
_NW = _NC * _NS
_BPW = B // _NW

@functools.cache
def _make_sc_gather():
    mesh = plsc.VectorSubcoreMesh(core_axis_name="c", subcore_axis_name="s",
                                  num_cores=_NC, num_subcores=_NS)

    @functools.partial(
        pl.kernel,
        out_type=(
            jax.ShapeDtypeStruct((B, D), jnp.float32),
            jax.ShapeDtypeStruct((B, D), jnp.float32),
        ),
        mesh=mesh,
        scratch_types=[
            pltpu.VMEM((_BPW,), jnp.int32),
            pltpu.VMEM((_BPW,), jnp.int32),
            pltpu.VMEM((_BPW, D), jnp.float32),
            pltpu.VMEM((_BPW, D), jnp.float32),
            pltpu.SemaphoreType.DMA,
            pltpu.SemaphoreType.DMA,
        ],
        compiler_params=pltpu.CompilerParams(use_tc_tiling_on_sc=False),
    )
    def _sc_gather(user_hbm, item_hbm, utab_hbm, itab_hbm, uout_hbm, iout_hbm,
                   uidx_v, iidx_v, urow_v, irow_v, usem, isem):
        wid = lax.axis_index("s") * _NC + lax.axis_index("c")
        base = wid * _BPW
        pltpu.sync_copy(user_hbm.at[pl.ds(base, _BPW)], uidx_v)
        pltpu.sync_copy(item_hbm.at[pl.ds(base, _BPW)], iidx_v)
        cu = pltpu.async_copy(utab_hbm.at[uidx_v], urow_v, usem)
        ci = pltpu.async_copy(itab_hbm.at[iidx_v], irow_v, isem)
        cu.wait()
        ci.wait()
        pltpu.sync_copy(urow_v, uout_hbm.at[pl.ds(base, _BPW)])
        pltpu.sync_copy(irow_v, iout_hbm.at[pl.ds(base, _BPW)])

    return _sc_gather


_BLK = 2048
_NBLK = B // _BLK


def _mlp_body(u_ref, i_ref, w1_ref, b1_ref, w2_ref, b2_ref, w3_ref, b3_ref,
              o_ref):
    dn = (((1,), (1,)), ((), ()))
    u = u_ref[...]
    v = i_ref[...]
    w1 = w1_ref[...]
    x = lax.dot_general(u, w1[:, :D], dn, preferred_element_type=jnp.float32)
    x = x + lax.dot_general(v, w1[:, D:], dn,
                            preferred_element_type=jnp.float32)
    x = jnp.maximum(x + b1_ref[...], 0.0)
    x = lax.dot_general(x, w2_ref[...], dn, preferred_element_type=jnp.float32)
    x = jnp.maximum(x + b2_ref[...], 0.0)
    y = lax.dot_general(w3_ref[...], x, dn, preferred_element_type=jnp.float32)
    o_ref[0] = y + b3_ref[...]


_mlp = pl.pallas_call(
    _mlp_body,
    grid=(_NBLK,),
    in_specs=[
        pl.BlockSpec((_BLK, D), lambda i: (i, 0)),
        pl.BlockSpec((_BLK, D), lambda i: (i, 0)),
        pl.BlockSpec((D, 2 * D), lambda i: (0, 0)),
        pl.BlockSpec((1, D), lambda i: (0, 0)),
        pl.BlockSpec((D // 2, D), lambda i: (0, 0)),
        pl.BlockSpec((1, D // 2), lambda i: (0, 0)),
        pl.BlockSpec((1, D // 2), lambda i: (0, 0)),
        pl.BlockSpec((1, 1), lambda i: (0, 0)),
    ],
    out_specs=pl.BlockSpec((1, 1, _BLK), lambda i: (i, 0, 0)),
    out_shape=jax.ShapeDtypeStruct((_NBLK, 1, _BLK), jnp.float32),
)


def kernel(user, item, user_table, item_table, W1, b1, W2, b2, W3, b3):
    u_emb, i_emb = _make_sc_gather()(user, item, user_table, item_table)
    out2d = _mlp(u_emb, i_emb, W1, b1.reshape(1, D), W2,
                 b2.reshape(1, D // 2), W3, b3.reshape(1, 1))
    return out2d.reshape(-1)
